# baseline (speedup 1.0000x reference)
import functools

import jax
import jax.numpy as jnp
from jax import lax
from jax.experimental import pallas as pl
from jax.experimental.pallas import tpu as pltpu
from jax.experimental.pallas import tpu_sc as plsc

_NC, _NS = 2, 16
_NW = _NC * _NS
_B, _L, _V, _D = 4096, 200, 16, 16
_BW = _B // _NW
_LC = 10
_NCHUNK = _L // _LC
_NBUF = 4
_NBG = _BW // 16


def _fused_table_body(w_ref, tt_ref, b_ref, o_ref):
    o_ref[...] = (
        jnp.dot(w_ref[...], tt_ref[...], preferred_element_type=jnp.float32)
        + b_ref[...]
    )


def _make_fused_table_t(W, tableT, b2):
    return pl.pallas_call(
        _fused_table_body,
        out_shape=jax.ShapeDtypeStruct((_D, _V), jnp.float32),
    )(W, tableT, b2)


_sc_mesh = plsc.VectorSubcoreMesh(core_axis_name="c", subcore_axis_name="s")


@functools.partial(
    pl.kernel,
    out_type=jax.ShapeDtypeStruct((_L * _D, _B), jnp.float32),
    mesh=_sc_mesh,
    scratch_types=[
        pltpu.VMEM((_V * _D,), jnp.float32),
        pltpu.VMEM((_L, _BW), jnp.int32),
        pltpu.VMEM((_LC * _D, _BW), jnp.float32),
        pltpu.VMEM((_LC * _D, _BW), jnp.float32),
        pltpu.VMEM((_LC * _D, _BW), jnp.float32),
        pltpu.VMEM((_LC * _D, _BW), jnp.float32),
        pltpu.SemaphoreType.DMA,
        pltpu.SemaphoreType.DMA,
        pltpu.SemaphoreType.DMA,
        pltpu.SemaphoreType.DMA,
    ],
    compiler_params=pltpu.CompilerParams(needs_layout_passes=False),
)
def _sc_gather(fused_hbm, ids_hbm, out_hbm, fused_v, ids_v, buf0_v, buf1_v,
               buf2_v, buf3_v, sem0, sem1, sem2, sem3):
    wid = lax.axis_index("s") * _NC + lax.axis_index("c")
    col0 = wid * _BW
    pltpu.sync_copy(fused_hbm, fused_v)
    pltpu.sync_copy(ids_hbm.at[:, pl.ds(col0, _BW)], ids_v)

    bufs = (buf0_v, buf1_v, buf2_v, buf3_v)
    sems = (sem0, sem1, sem2, sem3)

    @pl.loop(0, _NCHUNK // _NBUF)
    def _pair(di):
        for half in range(_NBUF):
            ci = di * _NBUF + half
            buf_v = bufs[half]

            @pl.when(di > 0)
            def _drain(half=half, buf_v=buf_v):
                pltpu.make_async_copy(
                    out_hbm.at[pl.ds(0, _LC * _D), pl.ds(col0, _BW)],
                    buf_v,
                    sems[half],
                ).wait()

            @plsc.parallel_loop(0, _LC, unroll=1)
            def _pos(i, ci=ci, buf_v=buf_v):
                l = ci * _LC + i
                for bg in range(_NBG):
                    idsv = ids_v[l, pl.ds(bg * 16, 16)]
                    for d in range(_D):
                        col = plsc.load_gather(fused_v, [idsv + d * 16])
                        buf_v[i * _D + d, pl.ds(bg * 16, 16)] = col

            pltpu.async_copy(
                buf_v,
                out_hbm.at[
                    pl.ds(ci * (_LC * _D), _LC * _D), pl.ds(col0, _BW)
                ],
                sems[half],
            )

    for half in range(_NBUF):
        pltpu.make_async_copy(
            out_hbm.at[pl.ds(0, _LC * _D), pl.ds(col0, _BW)],
            bufs[half],
            sems[half],
        ).wait()


def kernel(input_ids, table, W, b):
    ids_t = input_ids.T.astype(jnp.int32)
    fused_t = _make_fused_table_t(W, table.T, b.reshape(_D, 1))
    out = _sc_gather(fused_t.reshape(_V * _D), ids_t)
    return out.reshape(_L, _D, _B).transpose(2, 0, 1)


# speedup vs baseline: 1.5130x; 1.5130x over previous
import functools

import jax
import jax.numpy as jnp
from jax import lax
from jax.experimental import pallas as pl
from jax.experimental.pallas import tpu as pltpu
from jax.experimental.pallas import tpu_sc as plsc

_NC, _NS = 2, 16
_NW = _NC * _NS
_B, _L, _V, _D = 4096, 200, 16, 16
_BW = _B // _NW
_LC = 20
_NCHUNK = _L // _LC
_NBG = _BW // 16


def _fused_table_body(w_ref, tt_ref, b_ref, o_ref):
    o_ref[...] = (
        jnp.dot(w_ref[...], tt_ref[...], preferred_element_type=jnp.float32)
        + b_ref[...]
    )


def _make_fused_table_t(W, tableT, b2):
    return pl.pallas_call(
        _fused_table_body,
        out_shape=jax.ShapeDtypeStruct((_D, _V), jnp.float32),
    )(W, tableT, b2)


_sc_mesh = plsc.VectorSubcoreMesh(core_axis_name="c", subcore_axis_name="s")


@functools.partial(
    pl.kernel,
    out_type=jax.ShapeDtypeStruct((_L * _D, _B), jnp.float32),
    mesh=_sc_mesh,
    scratch_types=[
        pltpu.VMEM((_V * _D,), jnp.float32),
        pltpu.VMEM((_L, _BW), jnp.int32),
        pltpu.VMEM((_LC * _D, _BW), jnp.float32),
        pltpu.VMEM((_LC * _D, _BW), jnp.float32),
        pltpu.SemaphoreType.DMA,
        pltpu.SemaphoreType.DMA,
    ],
    compiler_params=pltpu.CompilerParams(needs_layout_passes=False),
)
def _sc_gather(fused_hbm, ids_hbm, out_hbm, fused_v, ids_v, buf0_v, buf1_v,
               sem0, sem1):
    wid = lax.axis_index("s") * _NC + lax.axis_index("c")
    col0 = wid * _BW
    pltpu.sync_copy(fused_hbm, fused_v)
    pltpu.sync_copy(ids_hbm.at[:, pl.ds(col0, _BW)], ids_v)

    bufs = (buf0_v, buf1_v)
    sems = (sem0, sem1)

    @pl.loop(0, _NCHUNK // 2)
    def _pair(di):
        for half in range(2):
            ci = di * 2 + half
            buf_v = bufs[half]

            @pl.when(di > 0)
            def _drain(half=half, buf_v=buf_v):
                pltpu.make_async_copy(
                    out_hbm.at[pl.ds(0, _LC * _D), pl.ds(col0, _BW)],
                    buf_v,
                    sems[half],
                ).wait()

            @plsc.parallel_loop(0, _LC, unroll=1)
            def _pos(i, ci=ci, buf_v=buf_v):
                l = ci * _LC + i
                for bg in range(_NBG):
                    idsv = ids_v[l, pl.ds(bg * 16, 16)]
                    for d in range(_D):
                        col = plsc.load_gather(fused_v, [idsv + d * 16])
                        buf_v[i * _D + d, pl.ds(bg * 16, 16)] = col

            pltpu.async_copy(
                buf_v,
                out_hbm.at[
                    pl.ds(ci * (_LC * _D), _LC * _D), pl.ds(col0, _BW)
                ],
                sems[half],
            )

    for half in range(2):
        pltpu.make_async_copy(
            out_hbm.at[pl.ds(0, _LC * _D), pl.ds(col0, _BW)],
            bufs[half],
            sems[half],
        ).wait()


def kernel(input_ids, table, W, b):
    ids_t = input_ids.T.astype(jnp.int32)
    fused_t = _make_fused_table_t(W, table.T, b.reshape(_D, 1))
    out = _sc_gather(fused_t.reshape(_V * _D), ids_t)
    return out.reshape(_L, _D, _B).transpose(2, 0, 1)
